# baseline (speedup 1.0000x reference)
import functools

import jax
import jax.numpy as jnp
from jax import lax
from jax.experimental import pallas as pl
from jax.experimental.pallas import tpu as pltpu
from jax.experimental.pallas import tpu_sc as plsc

_NBUF = 2
_SEQ_PER_CHUNK = 2


@functools.lru_cache(maxsize=None)
def _build(rows_total, vocab, d, seq_len):
    info = plsc.get_sparse_core_info()
    nc, ns = info.num_cores, info.num_subcores
    nw = nc * ns
    rpw = rows_total // nw
    chunk = _SEQ_PER_CHUNK * seq_len
    assert rpw % chunk == 0
    chunks = rpw // chunk

    mesh = plsc.VectorSubcoreMesh(core_axis_name="c", subcore_axis_name="s")

    @functools.partial(
        pl.kernel,
        mesh=mesh,
        compiler_params=pltpu.CompilerParams(use_tc_tiling_on_sc=False),
        out_type=jax.ShapeDtypeStruct((rows_total, d), jnp.float32),
        scratch_types=[
            pltpu.VMEM((_NBUF, chunk), jnp.int32),
            pltpu.VMEM((_NBUF, chunk, d), jnp.float32),
            pltpu.VMEM_SHARED((chunk, d), jnp.float32),
            pltpu.SemaphoreType.DMA,
            pltpu.SemaphoreType.DMA,
            pltpu.SemaphoreType.DMA,
            pltpu.SemaphoreType.DMA,
        ],
    )
    def emb(idx_hbm, tok_hbm, pos_hbm, out_hbm, idx_v, rows_v, pos_v,
            gsem0, gsem1, ssem0, ssem1):
        gsems = [gsem0, gsem1]
        ssems = [ssem0, ssem1]
        wid = lax.axis_index("s") * nc + lax.axis_index("c")
        base = wid * rpw

        @pl.when(lax.axis_index("s") == 0)
        def _():
            for s in range(_SEQ_PER_CHUNK):
                pltpu.sync_copy(pos_hbm, pos_v.at[pl.ds(s * seq_len, seq_len)])

        plsc.subcore_barrier()

        def fire_gather(g, b):
            pltpu.sync_copy(pos_v, rows_v.at[b])
            pltpu.sync_copy(idx_hbm.at[pl.ds(base + g * chunk, chunk)],
                            idx_v.at[b])
            pltpu.async_copy(tok_hbm.at[idx_v.at[b]], rows_v.at[b], gsems[b],
                             add=True)

        def wait_store(b):
            pltpu.make_async_copy(rows_v.at[b], out_hbm.at[pl.ds(0, chunk)],
                                  ssems[b]).wait()

        fire_gather(0, 0)

        def super_body(t, carry):
            for b in range(_NBUF):
                g = t * _NBUF + b
                nb = (b + 1) % _NBUF

                @pl.when(g >= 1)
                def _():
                    wait_store(nb)

                @pl.when(g + 1 < chunks)
                def _():
                    fire_gather(g + 1, nb)

                pltpu.make_async_copy(tok_hbm.at[idx_v.at[b]], rows_v.at[b],
                                      gsems[b]).wait()
                pltpu.async_copy(rows_v.at[b],
                                 out_hbm.at[pl.ds(base + g * chunk, chunk)],
                                 ssems[b])
            return carry

        lax.fori_loop(0, chunks // _NBUF, super_body, 0)
        wait_store((chunks - 1) % _NBUF)

    return emb


def kernel(x, token_table, pos_table):
    batch, seq_len = x.shape
    vocab, d = token_table.shape
    rows_total = batch * seq_len
    idx = x.reshape(rows_total).astype(jnp.int32)
    emb = _build(rows_total, vocab, d, seq_len)
    out = emb(idx, token_table.astype(jnp.float32), pos_table.astype(jnp.float32))
    return out.reshape(batch, seq_len, d)


# speedup vs baseline: 1.0720x; 1.0720x over previous
import functools

import jax
import jax.numpy as jnp
from jax import lax
from jax.experimental import pallas as pl
from jax.experimental.pallas import tpu as pltpu
from jax.experimental.pallas import tpu_sc as plsc

_NBUF = 4
_SEQ_PER_CHUNK = 2


@functools.lru_cache(maxsize=None)
def _build(rows_total, vocab, d, seq_len):
    info = plsc.get_sparse_core_info()
    nc, ns = info.num_cores, info.num_subcores
    nw = nc * ns
    rpw = rows_total // nw
    chunk = _SEQ_PER_CHUNK * seq_len
    assert rpw % (chunk * _NBUF) == 0
    chunks = rpw // chunk

    mesh = plsc.VectorSubcoreMesh(core_axis_name="c", subcore_axis_name="s")

    @functools.partial(
        pl.kernel,
        mesh=mesh,
        compiler_params=pltpu.CompilerParams(use_tc_tiling_on_sc=False),
        out_type=jax.ShapeDtypeStruct((rows_total, d), jnp.float32),
        scratch_types=[
            pltpu.VMEM((_NBUF, chunk), jnp.int32),
            pltpu.VMEM((_NBUF, chunk, d), jnp.float32),
            pltpu.VMEM_SHARED((chunk, d), jnp.float32),
            [pltpu.SemaphoreType.DMA] * _NBUF,
            [pltpu.SemaphoreType.DMA] * _NBUF,
            [pltpu.SemaphoreType.DMA] * _NBUF,
        ],
    )
    def emb(idx_hbm, tok_hbm, pos_hbm, out_hbm, idx_v, rows_v, pos_v,
            gsems, ssems, isems):
        wid = lax.axis_index("s") * nc + lax.axis_index("c")
        base = wid * rpw

        @pl.when(lax.axis_index("s") == 0)
        def _():
            for s in range(_SEQ_PER_CHUNK):
                pltpu.sync_copy(pos_hbm, pos_v.at[pl.ds(s * seq_len, seq_len)])

        plsc.subcore_barrier()

        def fire_idx(g, b):
            pltpu.async_copy(idx_hbm.at[pl.ds(base + g * chunk, chunk)],
                             idx_v.at[b], isems[b])

        def wait_idx(b):
            pltpu.make_async_copy(idx_hbm.at[pl.ds(0, chunk)], idx_v.at[b],
                                  isems[b]).wait()

        def fire_prefill(b):
            pltpu.async_copy(pos_v, rows_v.at[b], gsems[b])

        def wait_prefill(b):
            pltpu.make_async_copy(pos_v, rows_v.at[b], gsems[b]).wait()

        def fire_gather(g, b):
            pltpu.async_copy(tok_hbm.at[idx_v.at[b]], rows_v.at[b], gsems[b],
                             add=True)

        def wait_gather(b):
            pltpu.make_async_copy(tok_hbm.at[idx_v.at[b]], rows_v.at[b],
                                  gsems[b]).wait()

        def fire_store(g, b):
            pltpu.async_copy(rows_v.at[b],
                             out_hbm.at[pl.ds(base + g * chunk, chunk)],
                             ssems[b])

        def wait_store(b):
            pltpu.make_async_copy(rows_v.at[b], out_hbm.at[pl.ds(0, chunk)],
                                  ssems[b]).wait()

        fire_idx(0, 0)
        fire_idx(1, 1)
        fire_prefill(0)
        fire_prefill(1)
        wait_idx(0)
        wait_prefill(0)
        fire_gather(0, 0)

        def super_body(t, carry):
            for b in range(_NBUF):
                g = t * _NBUF + b
                b1 = (b + 1) % _NBUF
                b2 = (b + 2) % _NBUF

                @pl.when(g + 1 < chunks)
                def _():
                    wait_idx(b1)
                    wait_prefill(b1)
                    fire_gather(g + 1, b1)

                @pl.when(g + 2 < chunks)
                def _():
                    fire_idx(g + 2, b2)

                wait_gather(b)
                fire_store(g, b)

                @pl.when(g + 2 < chunks)
                def _():
                    @pl.when(g >= 2)
                    def _():
                        wait_store(b2)

                    fire_prefill(b2)
            return carry

        lax.fori_loop(0, chunks // _NBUF, super_body, 0)
        for b in range(_NBUF):
            wait_store(b)

    return emb


def kernel(x, token_table, pos_table):
    batch, seq_len = x.shape
    vocab, d = token_table.shape
    rows_total = batch * seq_len
    idx = x.reshape(rows_total).astype(jnp.int32)
    emb = _build(rows_total, vocab, d, seq_len)
    out = emb(idx, token_table.astype(jnp.float32), pos_table.astype(jnp.float32))
    return out.reshape(batch, seq_len, d)
